# baseline (speedup 1.0000x reference)
import functools

import jax
import jax.numpy as jnp
from jax import lax
from jax.experimental import pallas as pl
from jax.experimental.pallas import tpu as pltpu
from jax.experimental.pallas import tpu_sc as plsc

D = 64
SCALE = 8.0
NB = 4096
NT = 200
NC = 2
NS = 16
NW = NC * NS
C = 128
NBUF = 2


def _sc_embed(idxT, table):
    mesh = plsc.VectorSubcoreMesh(core_axis_name="c", subcore_axis_name="s")

    scratch = [pltpu.VMEM((NT, C), jnp.int32)]
    scratch += [pltpu.VMEM((C, D), jnp.float32) for _ in range(NBUF)]
    scratch += [pltpu.VMEM((8, 8, C), jnp.float32) for _ in range(NBUF)]
    scratch += [pltpu.SemaphoreType.DMA for _ in range(2 * NBUF + 1)]

    @functools.partial(
        pl.kernel,
        mesh=mesh,
        out_type=jax.ShapeDtypeStruct((NT, 8, NW, 8, C), jnp.float32),
        scratch_types=scratch,
        compiler_params=pltpu.CompilerParams(
            use_tc_tiling_on_sc=False, needs_layout_passes=False),
    )
    def kern(idx_hbm, tab_hbm, out_hbm, idx_v, *bufs_and_sems):
        gbuf = bufs_and_sems[:NBUF]
        sbuf = bufs_and_sems[NBUF:2 * NBUF]
        gsem = bufs_and_sems[2 * NBUF:3 * NBUF]
        ssem = bufs_and_sems[3 * NBUF:4 * NBUF]
        isem = bufs_and_sems[4 * NBUF]

        wid = lax.axis_index("s") * NC + lax.axis_index("c")
        pltpu.async_copy(
            idx_hbm.at[:, pl.ds(wid * C, C)], idx_v, isem).wait()

        iotas = [
            lax.broadcasted_iota(jnp.int32, (16,), 0) + 16 * k
            for k in range(8)
        ]

        def gather(j, p):
            pltpu.async_copy(tab_hbm.at[idx_v.at[j]], gbuf[p], gsem[p])

        def store(j, p):
            pltpu.async_copy(sbuf[p], out_hbm.at[j, :, wid], ssem[p])

        for p in range(NBUF):
            gather(p, p)

        def unit(i, _):
            j = i * NBUF
            for p in range(NBUF):
                t = j + p
                pltpu.make_async_copy(
                    tab_hbm.at[idx_v.at[t]], gbuf[p], gsem[p]).wait()

                @pl.when(t >= NBUF)
                def _():
                    pltpu.make_async_copy(
                        sbuf[p], out_hbm.at[0, :, wid], ssem[p]).wait()

                for dt in range(8):
                    for di in range(8):
                        dvec = jnp.full((16,), 8 * dt + di, jnp.int32)
                        for k in range(8):
                            v = plsc.load_gather(
                                gbuf[p], [iotas[k], dvec])
                            sbuf[p][dt, di, pl.ds(16 * k, 16)] = v * SCALE

                store(t, p)

                @pl.when(t + NBUF < NT)
                def _():
                    gather(t + NBUF, p)
            return 0

        lax.fori_loop(0, NT // NBUF, unit, 0)

        for p in range(NBUF):
            pltpu.make_async_copy(
                sbuf[p], out_hbm.at[0, :, wid], ssem[p]).wait()

    return kern(idxT, table)


def kernel(inp, emb_weight):
    idxT = inp.T
    out5 = _sc_embed(idxT, emb_weight)
    return out5.transpose(2, 4, 0, 1, 3).reshape(NB, NT, D)


# speedup vs baseline: 2.1643x; 2.1643x over previous
import functools

import jax
import jax.numpy as jnp
from jax import lax
from jax.experimental import pallas as pl
from jax.experimental.pallas import tpu as pltpu
from jax.experimental.pallas import tpu_sc as plsc

D = 64
SCALE = 8.0
NB = 4096
NT = 200
B = NB * NT
NC = 2
NS = 16
NW = NC * NS
BPW = B // NW
C = 128
NCHUNK = BPW // C
NBUF = 2


def _sc_gather(idx3, table):
    mesh = plsc.VectorSubcoreMesh(core_axis_name="c", subcore_axis_name="s")

    scratch = [pltpu.VMEM((NCHUNK, C), jnp.int32)]
    scratch += [pltpu.VMEM((C, D), jnp.float32) for _ in range(NBUF)]
    scratch += [pltpu.VMEM((C, 2 * D), jnp.float32) for _ in range(NBUF)]
    scratch += [pltpu.SemaphoreType.DMA for _ in range(2 * NBUF + 1)]

    @functools.partial(
        pl.kernel,
        mesh=mesh,
        out_type=jax.ShapeDtypeStruct((B, 2 * D), jnp.float32),
        scratch_types=scratch,
        compiler_params=pltpu.CompilerParams(
            use_tc_tiling_on_sc=False, needs_layout_passes=False),
    )
    def kern(idx_hbm, tab_hbm, out_hbm, idx_v, *bufs_and_sems):
        gbuf = bufs_and_sems[:NBUF]
        sbuf = bufs_and_sems[NBUF:2 * NBUF]
        gsem = bufs_and_sems[2 * NBUF:3 * NBUF]
        ssem = bufs_and_sems[3 * NBUF:4 * NBUF]
        isem = bufs_and_sems[4 * NBUF]

        wid = lax.axis_index("s") * NC + lax.axis_index("c")
        pltpu.async_copy(idx_hbm.at[wid], idx_v, isem).wait()

        def gather(ci, p):
            pltpu.async_copy(tab_hbm.at[idx_v.at[ci]], gbuf[p], gsem[p])

        for p in range(NBUF):
            gather(p, p)

        def outer(i, _):
            cg = i * NBUF
            for p in range(NBUF):
                ci = cg + p
                pltpu.make_async_copy(
                    tab_hbm.at[idx_v.at[ci]], gbuf[p], gsem[p]).wait()

                @pl.when(ci >= NBUF)
                def _():
                    pltpu.make_async_copy(
                        sbuf[p].at[:, pl.ds(0, D)],
                        out_hbm.at[pl.ds(0, C), pl.ds(0, D)],
                        ssem[p]).wait()

                for r in range(C):
                    for c in range(D // 16):
                        sl = pl.ds(c * 16, 16)
                        sbuf[p][r, sl] = gbuf[p][r, sl] * SCALE

                row = wid * BPW + ci * C
                pltpu.async_copy(
                    sbuf[p].at[:, pl.ds(0, D)],
                    out_hbm.at[pl.ds(row, C), pl.ds(0, D)],
                    ssem[p])

                @pl.when(ci + NBUF < NCHUNK)
                def _():
                    gather(ci + NBUF, p)
            return 0

        lax.fori_loop(0, NCHUNK // NBUF, outer, 0)

        for p in range(NBUF):
            pltpu.make_async_copy(
                sbuf[p].at[:, pl.ds(0, D)],
                out_hbm.at[pl.ds(0, C), pl.ds(0, D)], ssem[p]).wait()

    return kern(idx3, table)


def kernel(inp, emb_weight):
    idx3 = inp.reshape(NW, NCHUNK, C)
    out128 = _sc_gather(idx3, emb_weight)
    return out128.reshape(NB, NT, 2 * D)[:, :, :D]
